# baseline (speedup 1.0000x reference)
import jax
import jax.numpy as jnp
from jax import lax
from jax.experimental import pallas as pl
from jax.experimental.pallas import tpu as pltpu
from jax.experimental.pallas import tpu_sc as plsc

_NC = 2
_NS = 16
_L = 16
_NW = _NC * _NS

_BATCH = 16384
_K = 64
_BPW = _BATCH // _NW
_V = 1000000
_VPAD = ((_V + 127) // 128) * 128
_BLK = 128
_BCLAMP = _VPAD - _BLK
_NG = _BPW // _L
_RING = 6
_NPAIR = _BPW // 2

_i32 = jnp.int32


def _splat(x):
  return jnp.zeros((_L,), _i32) + x


def _gather_body(su_hbm, si_hbm, gut_hbm, git_hbm,
                 pu_out, pi_out,
                 idx_v, b_v, n_v, d_v, dist_v,
                 rb0, rb1, rb2, rb3, rb4, rb5, pairs_v,
                 sem0, sem1, sem2, sem3, sem4, sem5, sem_out):
  wid = lax.axis_index("s") * _NC + lax.axis_index("c")
  base = wid * _BPW
  lanes = lax.iota(_i32, _L)
  bufs = (rb0, rb1, rb2, rb3, rb4, rb5)
  sems = (sem0, sem1, sem2, sem3, sem4, sem5)

  def phase(src_idx_hbm, tbl_hbm, stage_out_hbm):
    pltpu.sync_copy(src_idx_hbm.at[pl.ds(base, _BPW)], idx_v)

    def prep(g, running):
      v = idx_v[pl.ds(g * _L, _L)]
      blk = jnp.minimum((v // _BLK) * _BLK, _BCLAMP)
      b_v[pl.ds(g * _L, _L)] = blk
      pos = g * _L + lanes
      prev_pos = jnp.maximum(pos - 1, 0)
      prev = plsc.load_gather(b_v, [prev_pos])
      is_new = jnp.where((blk != prev) | (pos == 0), 1, 0).astype(_i32)
      n_v[pl.ds(g * _L, _L)] = is_new
      cs = plsc.cumsum(is_new)
      d = running + cs - 1
      d_v[pl.ds(g * _L, _L)] = d
      plsc.store_scatter(dist_v, [d], blk, mask=is_new == 1)
      return running + cs[_L - 1]

    num_d = lax.fori_loop(0, _NG, prep, jnp.zeros((), _i32))

    for dd in range(_RING - 1):
      @pl.when(dd < num_d)
      def _():
        blkc = plsc.load_gather(dist_v, [_splat(dd)])[0]
        blkc = pl.multiple_of(blkc, 128)
        pltpu.async_copy(tbl_hbm.at[:, pl.ds(blkc, _BLK)], bufs[dd], sems[dd])

    def group(g, _):
      rg = idx_v[pl.ds(g * _L, _L)]
      bg = b_v[pl.ds(g * _L, _L)]
      ng = n_v[pl.ds(g * _L, _L)]
      dg = d_v[pl.ds(g * _L, _L)]
      for jj in range(_L):
        p = g * _L + jj
        r = rg[jj]
        d = dg[jj]
        slot = d % _RING

        @pl.when(ng[jj] == 1)
        def _():
          d_ahead = d + _RING - 1

          @pl.when(d_ahead < num_d)
          def _():
            nb = plsc.load_gather(dist_v, [_splat(d_ahead)])[0]
            nb = pl.multiple_of(nb, 128)
            slot_a = d_ahead % _RING
            for s in range(_RING):
              @pl.when(slot_a == s)
              def _():
                pltpu.async_copy(
                    tbl_hbm.at[:, pl.ds(nb, _BLK)], bufs[s], sems[s])

          for s in range(_RING):
            @pl.when(slot == s)
            def _():
              pltpu.make_async_copy(
                  tbl_hbm.at[:, pl.ds(0, _BLK)], bufs[s], sems[s]).wait()

        c = r - bg[jj]
        q = p // 2
        h = p % 2
        for s in range(_RING):
          @pl.when(slot == s)
          def _():
            for kq in range(_K // _L):
              krow = kq * _L + lanes
              col = plsc.load_gather(bufs[s], [krow, _splat(c)])
              pairs_v[q, pl.ds(h * _K + kq * _L, _L)] = col
      return _

    lax.fori_loop(0, _NG, group, None)
    pltpu.async_copy(
        pairs_v, stage_out_hbm.at[pl.ds(wid * _NPAIR, _NPAIR)], sem_out
    ).wait()

  phase(su_hbm, gut_hbm, pu_out)
  phase(si_hbm, git_hbm, pi_out)


def _k2_body(pu_hbm, pi_hbm, invu_hbm, invi_hbm,
             xui_hbm, gu_out_hbm, gi_out_hbm,
             invu_v, invi_v, pidxu_v, pidxi_v, paru_v, pari_v,
             rows_u0, rows_u1, rows_i0, rows_i1,
             uflat_v, iflat_v,
             blku0, blku1, blki0, blki1, xacc0, xacc1,
             sem_u0, sem_u1, sem_i0, sem_i1,
             sem_a0, sem_a1, sem_b0, sem_b1, sem_x0, sem_x1):
  wid = lax.axis_index("s") * _NC + lax.axis_index("c")
  base = wid * _BPW
  lanes = lax.iota(_i32, _L)
  _P = 128
  _NP = _BPW // _P

  rows_u = (rows_u0, rows_u1)
  rows_i = (rows_i0, rows_i1)
  blku = (blku0, blku1)
  blki = (blki0, blki1)
  xacc = (xacc0, xacc1)
  sem_u = (sem_u0, sem_u1)
  sem_i = (sem_i0, sem_i1)
  sem_a = (sem_a0, sem_a1)
  sem_b = (sem_b0, sem_b1)
  sem_x = (sem_x0, sem_x1)

  pltpu.sync_copy(invu_hbm.at[pl.ds(base, _BPW)], invu_v)
  pltpu.sync_copy(invi_hbm.at[pl.ds(base, _BPW)], invi_v)

  for cch in range(_BPW // _L):
    vu = invu_v[pl.ds(cch * _L, _L)]
    vi = invi_v[pl.ds(cch * _L, _L)]
    pidxu_v[pl.ds(cch * _L, _L)] = vu // 2
    pidxi_v[pl.ds(cch * _L, _L)] = vi // 2
    paru_v[pl.ds(cch * _L, _L)] = (vu % 2) * _K
    pari_v[pl.ds(cch * _L, _L)] = (vi % 2) * _K

  def fetch(p):
    s = p % 2
    pltpu.async_copy(
        pu_hbm.at[pidxu_v.at[pl.ds(p * _P, _P)]], rows_u[s], sem_u[s])
    pltpu.async_copy(
        pi_hbm.at[pidxi_v.at[pl.ds(p * _P, _P)]], rows_i[s], sem_i[s])

  fetch(0)
  for pass_ in range(_NP):
    s = pass_ % 2
    pb = base + pass_ * _P
    if pass_ + 1 < _NP:
      fetch(pass_ + 1)
    pltpu.make_async_copy(pu_hbm.at[pidxu_v.at[pl.ds(0, _P)]],
                          rows_u[s], sem_u[s]).wait()
    pltpu.make_async_copy(pi_hbm.at[pidxi_v.at[pl.ds(0, _P)]],
                          rows_i[s], sem_i[s]).wait()
    if pass_ >= 2:
      pltpu.make_async_copy(blku[s], gu_out_hbm.at[:, pl.ds(base, _P)],
                            sem_a[s]).wait()
      pltpu.make_async_copy(blki[s], gi_out_hbm.at[:, pl.ds(base, _P)],
                            sem_b[s]).wait()
      pltpu.make_async_copy(xacc[s], xui_hbm.at[pl.ds(base, _P)],
                            sem_x[s]).wait()

    for cch in range(_P // _L):
      xacc[s][pl.ds(cch * _L, _L)] = jnp.zeros((_L,), jnp.float32)

    def sel(cch, _):
      pu16 = paru_v[pl.ds(pass_ * _P + cch * _L, _L)]
      pi16 = pari_v[pl.ds(pass_ * _P + cch * _L, _L)]
      for jj in range(_L):
        j = cch * _L + jj
        pu_s = pu16[jj]
        pi_s = pi16[jj]
        for kq in range(_K // _L):
          uv = rows_u[s][j, pl.ds(pu_s + kq * _L, _L)]
          iv = rows_i[s][j, pl.ds(pi_s + kq * _L, _L)]
          uflat_v[pl.ds(j * (_K + 1) + kq * _L, _L)] = uv
          iflat_v[pl.ds(j * (_K + 1) + kq * _L, _L)] = iv
      return _

    lax.fori_loop(0, _P // _L, sel, None)

    def per_k(k, _):
      for cch in range(_P // _L):
        j16 = (cch * _L + lanes) * (_K + 1) + k
        uvec = plsc.load_gather(uflat_v, [j16])
        ivec = plsc.load_gather(iflat_v, [j16])
        blku[s][k, pl.ds(cch * _L, _L)] = uvec
        blki[s][k, pl.ds(cch * _L, _L)] = ivec
        xacc[s][pl.ds(cch * _L, _L)] = (
            xacc[s][pl.ds(cch * _L, _L)] + uvec * ivec)
      return _

    lax.fori_loop(0, _K, per_k, None)

    pltpu.async_copy(blku[s], gu_out_hbm.at[:, pl.ds(pb, _P)], sem_a[s])
    pltpu.async_copy(blki[s], gi_out_hbm.at[:, pl.ds(pb, _P)], sem_b[s])
    pltpu.async_copy(xacc[s], xui_hbm.at[pl.ds(pb, _P)], sem_x[s])

  for p in (_NP - 2, _NP - 1):
    s = p % 2
    pltpu.make_async_copy(blku[s], gu_out_hbm.at[:, pl.ds(base, _P)],
                          sem_a[s]).wait()
    pltpu.make_async_copy(blki[s], gi_out_hbm.at[:, pl.ds(base, _P)],
                          sem_b[s]).wait()
    pltpu.make_async_copy(xacc[s], xui_hbm.at[pl.ds(base, _P)],
                          sem_x[s]).wait()


@jax.jit
def kernel(users, items, Gu, Gi):
  mesh = plsc.VectorSubcoreMesh(
      core_axis_name="c", subcore_axis_name="s",
      num_cores=_NC, num_subcores=_NS)
  cparams = pltpu.CompilerParams(
      needs_layout_passes=False, use_tc_tiling_on_sc=True)

  gather_run = pl.kernel(
      _gather_body,
      out_type=(
          jax.ShapeDtypeStruct((_BATCH // 2, 128), jnp.float32),
          jax.ShapeDtypeStruct((_BATCH // 2, 128), jnp.float32),
      ),
      mesh=mesh,
      scratch_types=(
          pltpu.VMEM((_BPW,), _i32),
          pltpu.VMEM((_BPW,), _i32),
          pltpu.VMEM((_BPW,), _i32),
          pltpu.VMEM((_BPW,), _i32),
          pltpu.VMEM((_BPW,), _i32),
          pltpu.VMEM((_K, _BLK), jnp.float32),
          pltpu.VMEM((_K, _BLK), jnp.float32),
          pltpu.VMEM((_K, _BLK), jnp.float32),
          pltpu.VMEM((_K, _BLK), jnp.float32),
          pltpu.VMEM((_K, _BLK), jnp.float32),
          pltpu.VMEM((_K, _BLK), jnp.float32),
          pltpu.VMEM((_NPAIR, 128), jnp.float32),
      ) + (pltpu.SemaphoreType.DMA,) * 7,
      compiler_params=cparams,
  )

  k2_run = pl.kernel(
      _k2_body,
      out_type=(
          jax.ShapeDtypeStruct((_BATCH,), jnp.float32),
          jax.ShapeDtypeStruct((_K, _BATCH), jnp.float32),
          jax.ShapeDtypeStruct((_K, _BATCH), jnp.float32),
      ),
      mesh=mesh,
      scratch_types=(
          pltpu.VMEM((_BPW,), _i32),
          pltpu.VMEM((_BPW,), _i32),
          pltpu.VMEM((_BPW,), _i32),
          pltpu.VMEM((_BPW,), _i32),
          pltpu.VMEM((_BPW,), _i32),
          pltpu.VMEM((_BPW,), _i32),
          pltpu.VMEM((128, 128), jnp.float32),
          pltpu.VMEM((128, 128), jnp.float32),
          pltpu.VMEM((128, 128), jnp.float32),
          pltpu.VMEM((128, 128), jnp.float32),
          pltpu.VMEM((128 * (_K + 1),), jnp.float32),
          pltpu.VMEM((128 * (_K + 1),), jnp.float32),
          pltpu.VMEM((_K, 128), jnp.float32),
          pltpu.VMEM((_K, 128), jnp.float32),
          pltpu.VMEM((_K, 128), jnp.float32),
          pltpu.VMEM((_K, 128), jnp.float32),
          pltpu.VMEM((128,), jnp.float32),
          pltpu.VMEM((128,), jnp.float32),
      ) + (pltpu.SemaphoreType.DMA,) * 10,
      compiler_params=cparams,
  )

  ar = jnp.arange(_BATCH, dtype=_i32)
  su, _ou = lax.sort((users, ar), num_keys=1)
  si, _oi = lax.sort((items, ar), num_keys=1)
  inv_u = jnp.zeros((_BATCH,), _i32).at[_ou].set(ar)
  inv_i = jnp.zeros((_BATCH,), _i32).at[_oi].set(ar)

  pu, pi_ = gather_run(su, si, Gu.T, Gi.T)
  xui, gamma_ut, gamma_it = k2_run(pu, pi_, inv_u, inv_i)
  return (xui, gamma_ut.T, gamma_it.T)


# speedup vs baseline: 1.4129x; 1.4129x over previous
import jax
import jax.numpy as jnp
from jax import lax
from jax.experimental import pallas as pl
from jax.experimental.pallas import tpu as pltpu
from jax.experimental.pallas import tpu_sc as plsc

_NC = 2
_NS = 16
_L = 16
_NW = _NC * _NS

_BATCH = 16384
_K = 64
_BPW = _BATCH // _NW
_V = 1000000
_VPAD = ((_V + 127) // 128) * 128
_BLK = 128
_BCLAMP = _VPAD - _BLK
_NG = _BPW // _L
_RING = 4
_NPAIR = _BPW // 2

_i32 = jnp.int32


def _splat(x):
  return jnp.zeros((_L,), _i32) + x


def _gather_body(su_hbm, si_hbm, gut_hbm, git_hbm,
                 pu_out, pi_out,
                 idx_v, b_v, n_v, d_v, dist_v,
                 rb0, rb1, rb2, rb3, pairs_v,
                 sem0, sem1, sem2, sem3, sem_out):
  wid = lax.axis_index("s") * _NC + lax.axis_index("c")
  base = wid * _BPW
  lanes = lax.iota(_i32, _L)
  bufs = (rb0, rb1, rb2, rb3)
  sems = (sem0, sem1, sem2, sem3)

  def phase(src_idx_hbm, tbl_hbm, stage_out_hbm):
    pltpu.sync_copy(src_idx_hbm.at[pl.ds(base, _BPW)], idx_v)

    def prep(g, running):
      v = idx_v[pl.ds(g * _L, _L)]
      blk = jnp.minimum((v // _BLK) * _BLK, _BCLAMP)
      b_v[pl.ds(g * _L, _L)] = blk
      pos = g * _L + lanes
      prev_pos = jnp.maximum(pos - 1, 0)
      prev = plsc.load_gather(b_v, [prev_pos])
      is_new = jnp.where((blk != prev) | (pos == 0), 1, 0).astype(_i32)
      n_v[pl.ds(g * _L, _L)] = is_new
      cs = plsc.cumsum(is_new)
      d = running + cs - 1
      d_v[pl.ds(g * _L, _L)] = d
      plsc.store_scatter(dist_v, [d], blk, mask=is_new == 1)
      return running + cs[_L - 1]

    num_d = lax.fori_loop(0, _NG, prep, jnp.zeros((), _i32))

    for dd in range(_RING - 1):
      @pl.when(dd < num_d)
      def _():
        blkc = plsc.load_gather(dist_v, [_splat(dd)])[0]
        blkc = pl.multiple_of(blkc, 128)
        pltpu.async_copy(tbl_hbm.at[:, pl.ds(blkc, _BLK)], bufs[dd], sems[dd])

    def group(g, _):
      rg = idx_v[pl.ds(g * _L, _L)]
      bg = b_v[pl.ds(g * _L, _L)]
      ng = n_v[pl.ds(g * _L, _L)]
      dg = d_v[pl.ds(g * _L, _L)]
      for jj in range(_L):
        p = g * _L + jj
        r = rg[jj]
        d = dg[jj]
        slot = d % _RING

        @pl.when(ng[jj] == 1)
        def _():
          d_ahead = d + _RING - 1

          @pl.when(d_ahead < num_d)
          def _():
            nb = plsc.load_gather(dist_v, [_splat(d_ahead)])[0]
            nb = pl.multiple_of(nb, 128)
            slot_a = d_ahead % _RING
            for s in range(_RING):
              @pl.when(slot_a == s)
              def _():
                pltpu.async_copy(
                    tbl_hbm.at[:, pl.ds(nb, _BLK)], bufs[s], sems[s])

          for s in range(_RING):
            @pl.when(slot == s)
            def _():
              pltpu.make_async_copy(
                  tbl_hbm.at[:, pl.ds(0, _BLK)], bufs[s], sems[s]).wait()

        c = r - bg[jj]
        q = p // 2
        h = p % 2
        for s in range(_RING):
          @pl.when(slot == s)
          def _():
            for kq in range(_K // _L):
              krow = kq * _L + lanes
              col = plsc.load_gather(bufs[s], [krow, _splat(c)])
              pairs_v[q, pl.ds(h * _K + kq * _L, _L)] = col
      return _

    lax.fori_loop(0, _NG, group, None)
    pltpu.async_copy(
        pairs_v, stage_out_hbm.at[pl.ds(wid * _NPAIR, _NPAIR)], sem_out
    ).wait()

  phase(su_hbm, gut_hbm, pu_out)
  phase(si_hbm, git_hbm, pi_out)


def _k2_body(pu_hbm, pi_hbm, invu_hbm, invi_hbm,
             xui_hbm, gu_out_hbm, gi_out_hbm,
             invu_v, invi_v, pidxu_v, pidxi_v, paru_v, pari_v,
             rows_u0, rows_u1, rows_i0, rows_i1,
             uflat_v, iflat_v,
             blku0, blku1, blki0, blki1, xacc0, xacc1,
             sem_u0, sem_u1, sem_i0, sem_i1,
             sem_a0, sem_a1, sem_b0, sem_b1, sem_x0, sem_x1):
  wid = lax.axis_index("s") * _NC + lax.axis_index("c")
  base = wid * _BPW
  lanes = lax.iota(_i32, _L)
  _P = 128
  _NP = _BPW // _P

  rows_u = (rows_u0, rows_u1)
  rows_i = (rows_i0, rows_i1)
  blku = (blku0, blku1)
  blki = (blki0, blki1)
  xacc = (xacc0, xacc1)
  sem_u = (sem_u0, sem_u1)
  sem_i = (sem_i0, sem_i1)
  sem_a = (sem_a0, sem_a1)
  sem_b = (sem_b0, sem_b1)
  sem_x = (sem_x0, sem_x1)

  pltpu.sync_copy(invu_hbm.at[pl.ds(base, _BPW)], invu_v)
  pltpu.sync_copy(invi_hbm.at[pl.ds(base, _BPW)], invi_v)

  for cch in range(_BPW // _L):
    vu = invu_v[pl.ds(cch * _L, _L)]
    vi = invi_v[pl.ds(cch * _L, _L)]
    pidxu_v[pl.ds(cch * _L, _L)] = vu // 2
    pidxi_v[pl.ds(cch * _L, _L)] = vi // 2
    paru_v[pl.ds(cch * _L, _L)] = (vu % 2) * _K
    pari_v[pl.ds(cch * _L, _L)] = (vi % 2) * _K

  def fetch(p):
    s = p % 2
    pltpu.async_copy(
        pu_hbm.at[pidxu_v.at[pl.ds(p * _P, _P)]], rows_u[s], sem_u[s])
    pltpu.async_copy(
        pi_hbm.at[pidxi_v.at[pl.ds(p * _P, _P)]], rows_i[s], sem_i[s])

  fetch(0)
  for pass_ in range(_NP):
    s = pass_ % 2
    pb = base + pass_ * _P
    if pass_ + 1 < _NP:
      fetch(pass_ + 1)
    pltpu.make_async_copy(pu_hbm.at[pidxu_v.at[pl.ds(0, _P)]],
                          rows_u[s], sem_u[s]).wait()
    pltpu.make_async_copy(pi_hbm.at[pidxi_v.at[pl.ds(0, _P)]],
                          rows_i[s], sem_i[s]).wait()
    if pass_ >= 2:
      pltpu.make_async_copy(blku[s], gu_out_hbm.at[:, pl.ds(base, _P)],
                            sem_a[s]).wait()
      pltpu.make_async_copy(blki[s], gi_out_hbm.at[:, pl.ds(base, _P)],
                            sem_b[s]).wait()
      pltpu.make_async_copy(xacc[s], xui_hbm.at[pl.ds(base, _P)],
                            sem_x[s]).wait()

    for cch in range(_P // _L):
      xacc[s][pl.ds(cch * _L, _L)] = jnp.zeros((_L,), jnp.float32)

    def sel(cch, _):
      pu16 = paru_v[pl.ds(pass_ * _P + cch * _L, _L)]
      pi16 = pari_v[pl.ds(pass_ * _P + cch * _L, _L)]
      for jj in range(_L):
        j = cch * _L + jj
        pu_s = pu16[jj]
        pi_s = pi16[jj]
        for kq in range(_K // _L):
          uv = rows_u[s][j, pl.ds(pu_s + kq * _L, _L)]
          iv = rows_i[s][j, pl.ds(pi_s + kq * _L, _L)]
          uflat_v[pl.ds(j * (_K + 1) + kq * _L, _L)] = uv
          iflat_v[pl.ds(j * (_K + 1) + kq * _L, _L)] = iv
      return _

    lax.fori_loop(0, _P // _L, sel, None)

    def per_k(k, _):
      for cch in range(_P // _L):
        j16 = (cch * _L + lanes) * (_K + 1) + k
        uvec = plsc.load_gather(uflat_v, [j16])
        ivec = plsc.load_gather(iflat_v, [j16])
        blku[s][k, pl.ds(cch * _L, _L)] = uvec
        blki[s][k, pl.ds(cch * _L, _L)] = ivec
        xacc[s][pl.ds(cch * _L, _L)] = (
            xacc[s][pl.ds(cch * _L, _L)] + uvec * ivec)
      return _

    lax.fori_loop(0, _K, per_k, None)

    pltpu.async_copy(blku[s], gu_out_hbm.at[:, pl.ds(pb, _P)], sem_a[s])
    pltpu.async_copy(blki[s], gi_out_hbm.at[:, pl.ds(pb, _P)], sem_b[s])
    pltpu.async_copy(xacc[s], xui_hbm.at[pl.ds(pb, _P)], sem_x[s])

  for p in (_NP - 2, _NP - 1):
    s = p % 2
    pltpu.make_async_copy(blku[s], gu_out_hbm.at[:, pl.ds(base, _P)],
                          sem_a[s]).wait()
    pltpu.make_async_copy(blki[s], gi_out_hbm.at[:, pl.ds(base, _P)],
                          sem_b[s]).wait()
    pltpu.make_async_copy(xacc[s], xui_hbm.at[pl.ds(base, _P)],
                          sem_x[s]).wait()


@jax.jit
def kernel(users, items, Gu, Gi):
  mesh = plsc.VectorSubcoreMesh(
      core_axis_name="c", subcore_axis_name="s",
      num_cores=_NC, num_subcores=_NS)
  cparams = pltpu.CompilerParams(
      needs_layout_passes=False, use_tc_tiling_on_sc=True)

  gather_run = pl.kernel(
      _gather_body,
      out_type=(
          jax.ShapeDtypeStruct((_BATCH // 2, 128), jnp.float32),
          jax.ShapeDtypeStruct((_BATCH // 2, 128), jnp.float32),
      ),
      mesh=mesh,
      scratch_types=(
          pltpu.VMEM((_BPW,), _i32),
          pltpu.VMEM((_BPW,), _i32),
          pltpu.VMEM((_BPW,), _i32),
          pltpu.VMEM((_BPW,), _i32),
          pltpu.VMEM((_BPW,), _i32),
          pltpu.VMEM((_K, _BLK), jnp.float32),
          pltpu.VMEM((_K, _BLK), jnp.float32),
          pltpu.VMEM((_K, _BLK), jnp.float32),
          pltpu.VMEM((_K, _BLK), jnp.float32),
          pltpu.VMEM((_NPAIR, 128), jnp.float32),
          pltpu.SemaphoreType.DMA,
          pltpu.SemaphoreType.DMA,
          pltpu.SemaphoreType.DMA,
          pltpu.SemaphoreType.DMA,
          pltpu.SemaphoreType.DMA,
      ),
      compiler_params=cparams,
  )

  k2_run = pl.kernel(
      _k2_body,
      out_type=(
          jax.ShapeDtypeStruct((_BATCH,), jnp.float32),
          jax.ShapeDtypeStruct((_K, _BATCH), jnp.float32),
          jax.ShapeDtypeStruct((_K, _BATCH), jnp.float32),
      ),
      mesh=mesh,
      scratch_types=(
          pltpu.VMEM((_BPW,), _i32),
          pltpu.VMEM((_BPW,), _i32),
          pltpu.VMEM((_BPW,), _i32),
          pltpu.VMEM((_BPW,), _i32),
          pltpu.VMEM((_BPW,), _i32),
          pltpu.VMEM((_BPW,), _i32),
          pltpu.VMEM((128, 128), jnp.float32),
          pltpu.VMEM((128, 128), jnp.float32),
          pltpu.VMEM((128, 128), jnp.float32),
          pltpu.VMEM((128, 128), jnp.float32),
          pltpu.VMEM((128 * (_K + 1),), jnp.float32),
          pltpu.VMEM((128 * (_K + 1),), jnp.float32),
          pltpu.VMEM((_K, 128), jnp.float32),
          pltpu.VMEM((_K, 128), jnp.float32),
          pltpu.VMEM((_K, 128), jnp.float32),
          pltpu.VMEM((_K, 128), jnp.float32),
          pltpu.VMEM((128,), jnp.float32),
          pltpu.VMEM((128,), jnp.float32),
      ) + (pltpu.SemaphoreType.DMA,) * 10,
      compiler_params=cparams,
  )

  ar = jnp.arange(_BATCH, dtype=_i32)
  su, _ou = lax.sort((users, ar), num_keys=1)
  si, _oi = lax.sort((items, ar), num_keys=1)
  inv_u = jnp.zeros((_BATCH,), _i32).at[_ou].set(ar)
  inv_i = jnp.zeros((_BATCH,), _i32).at[_oi].set(ar)

  pu, pi_ = gather_run(su, si, Gu.T, Gi.T)
  xui, gamma_ut, gamma_it = k2_run(pu, pi_, inv_u, inv_i)
  return (xui, gamma_ut.T, gamma_it.T)
